# Initial kernel scaffold; baseline (speedup 1.0000x reference)
import jax
import jax.numpy as jnp
from jax.experimental import pallas as pl


def kernel(x, is_defect, edge_index, edge_attr, batch, atom_emb, defect_emb, gat_W, gat_att_src, gat_att_dst, gat_lin_edge, gat_att_edge, gat_bias, q_W, q_b, k_W, k_b, v_W, v_b, o_W, o_b, geo_W1, geo_b1, geo_W2, geo_b2, defect_bias, fc1_W, fc1_b, fc2_W, fc2_b):
    raise NotImplementedError("write your pallas kernel here")


# speedup vs baseline: 1.0768x; 1.0768x over previous
import math
import jax
import jax.numpy as jnp
from jax.experimental import pallas as pl

N = 10000
E = 320000
H = 128
NH = 4
HD = 32
NL = 3
NG = 32
BINS = 40


def _layer_norm(h):
    mu = jnp.mean(h, axis=-1, keepdims=True)
    var = jnp.var(h, axis=-1, keepdims=True)
    return (h - mu) / jnp.sqrt(var + 1e-5)


def _head_mlp_kernel(g_ref, w1_ref, b1_ref, w2_ref, b2_ref, o_ref):
    g = g_ref[...]
    z = jax.nn.silu(g @ w1_ref[...] + b1_ref[...][None, :])
    o_ref[...] = z @ w2_ref[...] + b2_ref[...][None, :]


def kernel(x, is_defect, edge_index, edge_attr, batch, atom_emb, defect_emb, gat_W, gat_att_src, gat_att_dst, gat_lin_edge, gat_att_edge, gat_bias, q_W, q_b, k_W, k_b, v_W, v_b, o_W, o_b, geo_W1, geo_b1, geo_W2, geo_b2, defect_bias, fc1_W, fc1_b, fc2_W, fc2_b):
    src = edge_index[0]
    dst = edge_index[1]
    h = atom_emb[x] + defect_emb[is_defect]
    centers = jnp.linspace(0.0, 8.0, BINS)
    edge_feat = jnp.exp(-10.0 * (edge_attr - centers[None, :]) ** 2)

    M = jnp.einsum("lbnd,lnd->lbn", gat_lin_edge.reshape(NL, BINS, NH, HD), gat_att_edge)
    a_e = jnp.einsum("eb,lbn->len", edge_feat, M)

    code = is_defect[src] * 2 + is_defect[dst]
    payload = []
    for l in range(NL):
        geo = jax.nn.silu(edge_feat @ geo_W1[l] + geo_b1[l]) @ geo_W2[l] + geo_b2[l]
        db = jnp.take(defect_bias[l], code, axis=1).T
        payload.append(geo + db)

    eps = 1e-16
    for l in range(NL):
        xp = (h @ gat_W[l]).reshape(N, NH, HD)
        a_s = (xp * gat_att_src[l][None]).sum(-1)
        a_d = (xp * gat_att_dst[l][None]).sum(-1)
        logit = a_s[src] + a_d[dst] + a_e[l]
        logit = jnp.where(logit > 0, logit, 0.2 * logit)
        e = jnp.exp(logit)
        num = jnp.zeros((N, NH, HD), h.dtype).at[dst].add(xp[src] * e[..., None])
        den = jnp.zeros((N, NH), h.dtype).at[dst].add(e)
        agg = num / (den + eps)[..., None]
        h = h + agg.reshape(N, H) + gat_bias[l]
        h = jax.nn.silu(_layer_norm(h))

    for l in range(NL):
        Q = (h @ q_W[l] + q_b[l]).reshape(N, NH, HD)
        K = (h @ k_W[l] + k_b[l]).reshape(N, NH, HD)
        V = (h @ v_W[l] + v_b[l]).reshape(N, NH, HD)
        score = (Q[src] * K[dst]).sum(-1) / math.sqrt(HD) + payload[l]
        e = jnp.exp(score)
        num = jnp.zeros((N, NH, HD), h.dtype).at[dst].add(V[src] * e[..., None])
        den = jnp.zeros((N, NH), h.dtype).at[dst].add(e)
        agg = num / (den + eps)[..., None]
        out = agg.reshape(N, H) @ o_W[l] + o_b[l]
        h = _layer_norm(h + out)

    cnt = jnp.zeros((NG,), h.dtype).at[batch].add(1.0)
    gsum = jnp.zeros((NG, H), h.dtype).at[batch].add(h)
    g = gsum / jnp.maximum(cnt, 1.0)[:, None]

    return pl.pallas_call(
        _head_mlp_kernel,
        out_shape=jax.ShapeDtypeStruct((NG, 1), jnp.float32),
    )(g, fc1_W, fc1_b, fc2_W, fc2_b)


# speedup vs baseline: 9.7433x; 9.0482x over previous
import functools
import math

import jax
import jax.numpy as jnp
from jax import lax
from jax.experimental import pallas as pl
from jax.experimental.pallas import tpu as pltpu
from jax.experimental.pallas import tpu_sc as plsc

N = 10000
E = 320000
H = 128
NH = 4
HD = 32
NL = 3
NG = 32
BINS = 40

NCORES = 2
NSUB = 16
NW = NCORES * NSUB
N_PAD = 10000
STRIPE = N_PAD // NSUB
EPW = 10240
E_PAD = EPW * NW
WSC = 144

_MESH = plsc.VectorSubcoreMesh(core_axis_name="c", subcore_axis_name="s")


def _iota16():
    return lax.iota(jnp.int32, 16)


def _splat(v):
    return jnp.full((16,), v, jnp.int32)


def _edge_kernel_body(mode, c_sz, tsrc, tdst, pay, srci, dsti, zrows, out,
                      agg, rows, kd, pay_v, e_buf, src_v, dst_v, sc_v,
                      sem, sem2):
    cid = lax.axis_index("c")
    sid = lax.axis_index("s")
    wid = cid * NSUB + sid
    nchunk = EPW // c_sz
    ngroup = c_sz // 16

    pltpu.sync_copy(zrows, agg.at[pl.ds(sid * STRIPE, STRIPE)])
    if mode == "attn":
        pltpu.sync_copy(zrows.at[pl.ds(0, c_sz)], sc_v)
    plsc.subcore_barrier()

    sc = rows if mode == "gat" else sc_v
    voff = 0 if mode == "gat" else H

    def chunk(ci, _):
        base = wid * EPW + ci * c_sz
        pltpu.sync_copy(srci.at[pl.ds(base, c_sz)], src_v)
        pltpu.sync_copy(dsti.at[pl.ds(base, c_sz)], dst_v)
        pltpu.sync_copy(pay.at[:, pl.ds(base, c_sz)], pay_v)
        pltpu.async_copy(tsrc.at[src_v], rows, sem).wait()
        pltpu.async_copy(tdst.at[dst_v], kd, sem2).wait()

        def group(g, _):
            g16 = g * 16 + _iota16()
            for h in range(NH):
                pay_h = pay_v[h, pl.ds(g * 16, 16)]
                if mode == "gat":
                    asrc = plsc.load_gather(rows, [g16, _splat(H + h)])
                    adst = plsc.load_gather(kd, [g16, _splat(h)])
                    logit = asrc + adst + pay_h
                    logit = jnp.maximum(logit, 0.2 * logit)
                    e_h = jnp.exp(logit)
                else:
                    def dotstep(j, acc):
                        col = _splat(h * HD) + j
                        qc = plsc.load_gather(rows, [g16, col])
                        kc = plsc.load_gather(kd, [g16, col])
                        return acc + qc * kc
                    acc = lax.fori_loop(0, HD, dotstep,
                                        jnp.zeros((16,), jnp.float32))
                    e_h = jnp.exp(acc * (1.0 / math.sqrt(HD)) + pay_h)
                plsc.store_scatter(sc, [g16, _splat(H + h)], e_h)
                e_buf[h, pl.ds(g * 16, 16)] = e_h
            return 0

        lax.fori_loop(0, ngroup, group, 0)

        def edge(cc, _):
            for hp in range(NH):
                es = plsc.load_gather(e_buf, [_splat(hp), _splat(cc)])
                for k in (2 * hp, 2 * hp + 1):
                    v = rows[cc, pl.ds(voff + k * 16, 16)]
                    sc[cc, pl.ds(k * 16, 16)] = v * es
            return 0

        lax.fori_loop(0, c_sz, edge, 0)

        pltpu.sync_copy(sc, agg.at[dst_v], add=True)
        return 0

    lax.fori_loop(0, nchunk, chunk, 0)
    plsc.subcore_barrier()
    pltpu.sync_copy(agg.at[pl.ds(sid * STRIPE, STRIPE)],
                    out.at[cid, pl.ds(sid * STRIPE, STRIPE)])


def _make_edge_kernel(mode, c_sz, wsrc, wdst):
    body = functools.partial(_edge_kernel_body, mode, c_sz)
    return pl.kernel(
        body,
        mesh=_MESH,
        out_type=jax.ShapeDtypeStruct((NCORES, N_PAD, WSC), jnp.float32),
        scratch_types=[
            pltpu.VMEM_SHARED((N_PAD, WSC), jnp.float32),
            pltpu.VMEM((c_sz, wsrc), jnp.float32),
            pltpu.VMEM((c_sz, wdst), jnp.float32),
            pltpu.VMEM((NH, c_sz), jnp.float32),
            pltpu.VMEM((NH, c_sz), jnp.float32),
            pltpu.VMEM((c_sz,), jnp.int32),
            pltpu.VMEM((c_sz,), jnp.int32),
            pltpu.VMEM((c_sz if mode == "attn" else 16, WSC), jnp.float32),
            pltpu.SemaphoreType.DMA,
            pltpu.SemaphoreType.DMA,
        ],
        compiler_params=pltpu.CompilerParams(use_tc_tiling_on_sc=False,
                                             needs_layout_passes=False),
    )


_gat_edge = _make_edge_kernel("gat", 128, WSC, 16)
_attn_edge = _make_edge_kernel("attn", 64, 2 * H, H)


def _layer_norm(h):
    mu = jnp.mean(h, axis=-1, keepdims=True)
    var = jnp.var(h, axis=-1, keepdims=True)
    return (h - mu) / jnp.sqrt(var + 1e-5)


def _head_mlp_kernel(g_ref, w1_ref, b1_ref, w2_ref, b2_ref, o_ref):
    g = g_ref[...]
    z = jax.nn.silu(g @ w1_ref[...] + b1_ref[...][None, :])
    o_ref[...] = z @ w2_ref[...] + b2_ref[...][None, :]


def _pad_rows(a, rows):
    return jnp.pad(a, ((0, rows - a.shape[0]), (0, 0)))


def kernel(x, is_defect, edge_index, edge_attr, batch, atom_emb, defect_emb, gat_W, gat_att_src, gat_att_dst, gat_lin_edge, gat_att_edge, gat_bias, q_W, q_b, k_W, k_b, v_W, v_b, o_W, o_b, geo_W1, geo_b1, geo_W2, geo_b2, defect_bias, fc1_W, fc1_b, fc2_W, fc2_b):
    src = edge_index[0]
    dst = edge_index[1]
    srci = jnp.pad(src, (0, E_PAD - E)).astype(jnp.int32)
    dsti = jnp.pad(dst, (0, E_PAD - E)).astype(jnp.int32)
    zrows = jnp.zeros((STRIPE, WSC), jnp.float32)

    h = atom_emb[x] + defect_emb[is_defect]
    centers = jnp.linspace(0.0, 8.0, BINS)
    edge_feat = jnp.exp(-10.0 * (edge_attr - centers[None, :]) ** 2)

    M = jnp.einsum("lbnd,lnd->lbn", gat_lin_edge.reshape(NL, BINS, NH, HD), gat_att_edge)
    a_e = jnp.einsum("eb,lbn->lne", edge_feat, M)
    code = is_defect[src] * 2 + is_defect[dst]
    pays = []
    for l in range(NL):
        geo = jax.nn.silu(edge_feat @ geo_W1[l] + geo_b1[l]) @ geo_W2[l] + geo_b2[l]
        db = jnp.take(defect_bias[l], code, axis=1)
        pays.append(geo.T + db)
    pay_attn = jnp.stack(pays)
    pad_cols = jnp.full((NL, NH, E_PAD - E), -1e30, jnp.float32)
    a_e = jnp.concatenate([a_e, pad_cols], axis=-1)
    pay_attn = jnp.concatenate([pay_attn, pad_cols], axis=-1)

    eps = 1e-16
    for l in range(NL):
        xp = h @ gat_W[l]
        a_s = (xp.reshape(N, NH, HD) * gat_att_src[l][None]).sum(-1)
        a_d = (xp.reshape(N, NH, HD) * gat_att_dst[l][None]).sum(-1)
        tsrc = _pad_rows(jnp.concatenate(
            [xp, a_s, jnp.zeros((N, WSC - H - NH), jnp.float32)], axis=1), N_PAD)
        tdst = _pad_rows(jnp.concatenate(
            [a_d, jnp.zeros((N, 16 - NH), jnp.float32)], axis=1), N_PAD)
        out = _gat_edge(tsrc, tdst, a_e[l], srci, dsti, zrows)
        acc = out[0] + out[1]
        num = acc[:N, :H]
        den = acc[:N, H:H + NH]
        aggv = num * jnp.repeat(1.0 / (den + eps), HD, axis=1)
        h = h + aggv + gat_bias[l]
        h = jax.nn.silu(_layer_norm(h))

    for l in range(NL):
        Q = h @ q_W[l] + q_b[l]
        K = h @ k_W[l] + k_b[l]
        V = h @ v_W[l] + v_b[l]
        tsrc = _pad_rows(jnp.concatenate([Q, V], axis=1), N_PAD)
        tdst = _pad_rows(K, N_PAD)
        out = _attn_edge(tsrc, tdst, pay_attn[l], srci, dsti, zrows)
        acc = out[0] + out[1]
        num = acc[:N, :H]
        den = acc[:N, H:H + NH]
        aggv = num * jnp.repeat(1.0 / (den + eps), HD, axis=1)
        h = _layer_norm(h + aggv @ o_W[l] + o_b[l])

    cnt = jnp.zeros((NG,), h.dtype).at[batch].add(1.0)
    gsum = jnp.zeros((NG, H), h.dtype).at[batch].add(h)
    g = gsum / jnp.maximum(cnt, 1.0)[:, None]

    return pl.pallas_call(
        _head_mlp_kernel,
        out_shape=jax.ShapeDtypeStruct((NG, 1), jnp.float32),
    )(g, fc1_W, fc1_b, fc2_W, fc2_b)


# speedup vs baseline: 10.5690x; 1.0847x over previous
import functools
import math

import jax
import jax.numpy as jnp
from jax import lax
from jax.experimental import pallas as pl
from jax.experimental.pallas import tpu as pltpu
from jax.experimental.pallas import tpu_sc as plsc

N = 10000
E = 320000
H = 128
NH = 4
HD = 32
NL = 3
NG = 32
BINS = 40

NCORES = 2
NSUB = 16
NW = NCORES * NSUB
N_PAD = 10000
STRIPE = N_PAD // NSUB
EPW = 10240
E_PAD = EPW * NW
WSC = 144

_MESH = plsc.VectorSubcoreMesh(core_axis_name="c", subcore_axis_name="s")


def _iota16():
    return lax.iota(jnp.int32, 16)


def _splat(v):
    return jnp.full((16,), v, jnp.int32)


def _edge_kernel_body(mode, c_sz, tsrc, tdst, pay, srci, dsti, zrows, out,
                      agg, rows, kd, pay_v, e_buf, src_v, dst_v, sc_v,
                      sem, sem2):
    cid = lax.axis_index("c")
    sid = lax.axis_index("s")
    wid = cid * NSUB + sid
    nchunk = EPW // c_sz
    ngroup = c_sz // 16

    pltpu.sync_copy(zrows, agg.at[pl.ds(sid * STRIPE, STRIPE)])
    if mode == "attn":
        pltpu.sync_copy(zrows.at[pl.ds(0, c_sz)], sc_v)
    plsc.subcore_barrier()

    sc = rows if mode == "gat" else sc_v
    voff = 0 if mode == "gat" else H

    def chunk(ci, _):
        base = wid * EPW + ci * c_sz
        pltpu.sync_copy(srci.at[pl.ds(base, c_sz)], src_v)
        pltpu.sync_copy(dsti.at[pl.ds(base, c_sz)], dst_v)
        pltpu.sync_copy(pay.at[:, pl.ds(base, c_sz)], pay_v)
        cp1 = pltpu.async_copy(tsrc.at[src_v], rows, sem)
        cp2 = pltpu.async_copy(tdst.at[dst_v], kd, sem2)
        cp1.wait()
        cp2.wait()

        for g in range(ngroup):
            g16 = g * 16 + _iota16()
            for h in range(NH):
                pay_h = pay_v[h, pl.ds(g * 16, 16)]
                if mode == "gat":
                    asrc = plsc.load_gather(rows, [g16, _splat(H + h)])
                    adst = plsc.load_gather(kd, [g16, _splat(h)])
                    logit = asrc + adst + pay_h
                    logit = jnp.maximum(logit, 0.2 * logit)
                    e_h = jnp.exp(logit)
                else:
                    acc = jnp.zeros((16,), jnp.float32)
                    for j in range(HD):
                        col = _splat(h * HD + j)
                        qc = plsc.load_gather(rows, [g16, col])
                        kc = plsc.load_gather(kd, [g16, col])
                        acc = acc + qc * kc
                    e_h = jnp.exp(acc * (1.0 / math.sqrt(HD)) + pay_h)
                plsc.store_scatter(sc, [g16, _splat(H + h)], e_h)
                e_buf[h, pl.ds(g * 16, 16)] = e_h

        def edge(c2, _):
            for u in range(2):
                cc = c2 * 2 + u
                for hp in range(NH):
                    es = plsc.load_gather(e_buf, [_splat(hp), _splat(cc)])
                    for k in (2 * hp, 2 * hp + 1):
                        v = rows[cc, pl.ds(voff + k * 16, 16)]
                        sc[cc, pl.ds(k * 16, 16)] = v * es
            return 0

        lax.fori_loop(0, c_sz // 2, edge, 0)

        pltpu.sync_copy(sc, agg.at[dst_v], add=True)
        return 0

    lax.fori_loop(0, nchunk, chunk, 0)
    plsc.subcore_barrier()
    pltpu.sync_copy(agg.at[pl.ds(sid * STRIPE, STRIPE)],
                    out.at[cid, pl.ds(sid * STRIPE, STRIPE)])


def _make_edge_kernel(mode, c_sz, wsrc, wdst):
    body = functools.partial(_edge_kernel_body, mode, c_sz)
    return pl.kernel(
        body,
        mesh=_MESH,
        out_type=jax.ShapeDtypeStruct((NCORES, N_PAD, WSC), jnp.float32),
        scratch_types=[
            pltpu.VMEM_SHARED((N_PAD, WSC), jnp.float32),
            pltpu.VMEM((c_sz, wsrc), jnp.float32),
            pltpu.VMEM((c_sz, wdst), jnp.float32),
            pltpu.VMEM((NH, c_sz), jnp.float32),
            pltpu.VMEM((NH, c_sz), jnp.float32),
            pltpu.VMEM((c_sz,), jnp.int32),
            pltpu.VMEM((c_sz,), jnp.int32),
            pltpu.VMEM((c_sz if mode == "attn" else 16, WSC), jnp.float32),
            pltpu.SemaphoreType.DMA,
            pltpu.SemaphoreType.DMA,
        ],
        compiler_params=pltpu.CompilerParams(use_tc_tiling_on_sc=False,
                                             needs_layout_passes=False),
    )


_gat_edge = _make_edge_kernel("gat", 128, WSC, 16)
_attn_edge = _make_edge_kernel("attn", 64, 2 * H, H)


def _layer_norm(h):
    mu = jnp.mean(h, axis=-1, keepdims=True)
    var = jnp.var(h, axis=-1, keepdims=True)
    return (h - mu) / jnp.sqrt(var + 1e-5)


def _head_mlp_kernel(g_ref, w1_ref, b1_ref, w2_ref, b2_ref, o_ref):
    g = g_ref[...]
    z = jax.nn.silu(g @ w1_ref[...] + b1_ref[...][None, :])
    o_ref[...] = z @ w2_ref[...] + b2_ref[...][None, :]


def _pad_rows(a, rows):
    return jnp.pad(a, ((0, rows - a.shape[0]), (0, 0)))


def kernel(x, is_defect, edge_index, edge_attr, batch, atom_emb, defect_emb, gat_W, gat_att_src, gat_att_dst, gat_lin_edge, gat_att_edge, gat_bias, q_W, q_b, k_W, k_b, v_W, v_b, o_W, o_b, geo_W1, geo_b1, geo_W2, geo_b2, defect_bias, fc1_W, fc1_b, fc2_W, fc2_b):
    src = edge_index[0]
    dst = edge_index[1]
    srci = jnp.pad(src, (0, E_PAD - E)).astype(jnp.int32)
    dsti = jnp.pad(dst, (0, E_PAD - E)).astype(jnp.int32)
    zrows = jnp.zeros((STRIPE, WSC), jnp.float32)

    h = atom_emb[x] + defect_emb[is_defect]
    centers = jnp.linspace(0.0, 8.0, BINS)
    edge_feat = jnp.exp(-10.0 * (edge_attr - centers[None, :]) ** 2)

    M = jnp.einsum("lbnd,lnd->lbn", gat_lin_edge.reshape(NL, BINS, NH, HD), gat_att_edge)
    a_e = jnp.einsum("eb,lbn->lne", edge_feat, M)
    code = is_defect[src] * 2 + is_defect[dst]
    pays = []
    for l in range(NL):
        geo = jax.nn.silu(edge_feat @ geo_W1[l] + geo_b1[l]) @ geo_W2[l] + geo_b2[l]
        db = jnp.take(defect_bias[l], code, axis=1)
        pays.append(geo.T + db)
    pay_attn = jnp.stack(pays)
    pad_cols = jnp.full((NL, NH, E_PAD - E), -1e30, jnp.float32)
    a_e = jnp.concatenate([a_e, pad_cols], axis=-1)
    pay_attn = jnp.concatenate([pay_attn, pad_cols], axis=-1)

    eps = 1e-16
    for l in range(NL):
        xp = h @ gat_W[l]
        a_s = (xp.reshape(N, NH, HD) * gat_att_src[l][None]).sum(-1)
        a_d = (xp.reshape(N, NH, HD) * gat_att_dst[l][None]).sum(-1)
        tsrc = _pad_rows(jnp.concatenate(
            [xp, a_s, jnp.zeros((N, WSC - H - NH), jnp.float32)], axis=1), N_PAD)
        tdst = _pad_rows(jnp.concatenate(
            [a_d, jnp.zeros((N, 16 - NH), jnp.float32)], axis=1), N_PAD)
        out = _gat_edge(tsrc, tdst, a_e[l], srci, dsti, zrows)
        acc = out[0] + out[1]
        num = acc[:N, :H]
        den = acc[:N, H:H + NH]
        aggv = num * jnp.repeat(1.0 / (den + eps), HD, axis=1)
        h = h + aggv + gat_bias[l]
        h = jax.nn.silu(_layer_norm(h))

    for l in range(NL):
        Q = h @ q_W[l] + q_b[l]
        K = h @ k_W[l] + k_b[l]
        V = h @ v_W[l] + v_b[l]
        tsrc = _pad_rows(jnp.concatenate([Q, V], axis=1), N_PAD)
        tdst = _pad_rows(K, N_PAD)
        out = _attn_edge(tsrc, tdst, pay_attn[l], srci, dsti, zrows)
        acc = out[0] + out[1]
        num = acc[:N, :H]
        den = acc[:N, H:H + NH]
        aggv = num * jnp.repeat(1.0 / (den + eps), HD, axis=1)
        h = _layer_norm(h + aggv @ o_W[l] + o_b[l])

    cnt = jnp.zeros((NG,), h.dtype).at[batch].add(1.0)
    gsum = jnp.zeros((NG, H), h.dtype).at[batch].add(h)
    g = gsum / jnp.maximum(cnt, 1.0)[:, None]

    return pl.pallas_call(
        _head_mlp_kernel,
        out_shape=jax.ShapeDtypeStruct((NG, 1), jnp.float32),
    )(g, fc1_W, fc1_b, fc2_W, fc2_b)


# speedup vs baseline: 11.2744x; 1.0667x over previous
import functools
import math

import jax
import jax.numpy as jnp
from jax import lax
from jax.experimental import pallas as pl
from jax.experimental.pallas import tpu as pltpu
from jax.experimental.pallas import tpu_sc as plsc

N = 10000
E = 320000
H = 128
NH = 4
HD = 32
NL = 3
NG = 32
BINS = 40

NCORES = 2
NSUB = 16
NW = NCORES * NSUB
N_PAD = 10000
STRIPE = N_PAD // NSUB
EPW = 10240
E_PAD = EPW * NW
WSC = 144

_MESH = plsc.VectorSubcoreMesh(core_axis_name="c", subcore_axis_name="s")


def _iota16():
    return lax.iota(jnp.int32, 16)


def _splat(v):
    return jnp.full((16,), v, jnp.int32)


def _edge_kernel_body(mode, c_sz, poff, tsrc, tdst, pay, srci, dsti, zrows, out,
                      agg, rows, kd, pay_v, e_buf, src_v, dst_v, sc_v,
                      sem, sem2):
    cid = lax.axis_index("c")
    sid = lax.axis_index("s")
    wid = cid * NSUB + sid
    nchunk = EPW // c_sz
    ngroup = c_sz // 16

    pltpu.sync_copy(zrows, agg.at[pl.ds(sid * STRIPE, STRIPE)])
    if mode == "attn":
        pltpu.sync_copy(zrows.at[pl.ds(0, c_sz)], sc_v)
    plsc.subcore_barrier()

    sc = rows if mode == "gat" else sc_v
    voff = 0 if mode == "gat" else H

    def chunk(ci, _):
        base = wid * EPW + ci * c_sz
        pltpu.sync_copy(srci.at[pl.ds(base, c_sz)], src_v)
        pltpu.sync_copy(dsti.at[pl.ds(base, c_sz)], dst_v)
        pltpu.sync_copy(pay.at[:, pl.ds(base, c_sz)], pay_v)
        cp1 = pltpu.async_copy(tsrc.at[src_v], rows, sem)
        cp2 = pltpu.async_copy(tdst.at[dst_v], kd, sem2)
        cp1.wait()
        cp2.wait()

        for g in range(ngroup):
            g16 = g * 16 + _iota16()
            for h in range(NH):
                pay_h = pay_v[poff + h, pl.ds(g * 16, 16)]
                if mode == "gat":
                    asrc = plsc.load_gather(rows, [g16, _splat(H + h)])
                    adst = plsc.load_gather(kd, [g16, _splat(h)])
                    logit = asrc + adst + pay_h
                    logit = jnp.maximum(logit, 0.2 * logit)
                    e_h = jnp.exp(logit)
                else:
                    acc = jnp.zeros((16,), jnp.float32)
                    for j in range(HD):
                        col = _splat(h * HD + j)
                        qc = plsc.load_gather(rows, [g16, col])
                        kc = plsc.load_gather(kd, [g16, col])
                        acc = acc + qc * kc
                    e_h = jnp.exp(acc * (1.0 / math.sqrt(HD)) + pay_h)
                plsc.store_scatter(sc, [g16, _splat(H + h)], e_h)
                e_buf[h, pl.ds(g * 16, 16)] = e_h

        def edge(c2, _):
            for u in range(2):
                cc = c2 * 2 + u
                for hp in range(NH):
                    es = plsc.load_gather(e_buf, [_splat(hp), _splat(cc)])
                    for k in (2 * hp, 2 * hp + 1):
                        v = rows[cc, pl.ds(voff + k * 16, 16)]
                        sc[cc, pl.ds(k * 16, 16)] = v * es
            return 0

        lax.fori_loop(0, c_sz // 2, edge, 0)

        pltpu.sync_copy(sc, agg.at[dst_v], add=True)
        return 0

    lax.fori_loop(0, nchunk, chunk, 0)
    plsc.subcore_barrier()
    pltpu.sync_copy(agg.at[pl.ds(sid * STRIPE, STRIPE)],
                    out.at[cid, pl.ds(sid * STRIPE, STRIPE)])


def _make_edge_kernel(mode, c_sz, poff, wsrc, wdst):
    body = functools.partial(_edge_kernel_body, mode, c_sz, poff)
    return pl.kernel(
        body,
        mesh=_MESH,
        out_type=jax.ShapeDtypeStruct((NCORES, N_PAD, WSC), jnp.float32),
        scratch_types=[
            pltpu.VMEM_SHARED((N_PAD, WSC), jnp.float32),
            pltpu.VMEM((c_sz, wsrc), jnp.float32),
            pltpu.VMEM((c_sz, wdst), jnp.float32),
            pltpu.VMEM((8, c_sz), jnp.float32),
            pltpu.VMEM((NH, c_sz), jnp.float32),
            pltpu.VMEM((c_sz,), jnp.int32),
            pltpu.VMEM((c_sz,), jnp.int32),
            pltpu.VMEM((c_sz if mode == "attn" else 16, WSC), jnp.float32),
            pltpu.SemaphoreType.DMA,
            pltpu.SemaphoreType.DMA,
        ],
        compiler_params=pltpu.CompilerParams(use_tc_tiling_on_sc=False,
                                             needs_layout_passes=False),
    )


_gat_edge = _make_edge_kernel("gat", 128, 0, WSC, 16)
_attn_edge = _make_edge_kernel("attn", 64, 4, 2 * H, H)


ET = 4096
NBLK = E_PAD // ET
NT = 2000
NNB = N_PAD // NT


def _payload_tc_body(attr_ref, oh_ref, mt_ref, w1t_ref, b1_ref, w2t_ref,
                     b2_ref, db_ref, out_ref):
    attr = attr_ref[0, 0, :]
    cent = lax.broadcasted_iota(jnp.int32, (BINS, 1), 0).astype(jnp.float32) * (
        8.0 / (BINS - 1))
    ef_t = jnp.exp(-10.0 * (attr[None, :] - cent) ** 2)
    a_et = mt_ref[0] @ ef_t
    z = w1t_ref[0] @ ef_t + b1_ref[0, 0][:, None]
    geo_t = w2t_ref[0] @ jax.nn.silu(z) + b2_ref[0, 0][:, None]
    db_t = db_ref[0] @ oh_ref[...]
    out_ref[0, 0:NH, :] = a_et
    out_ref[0, NH:2 * NH, :] = geo_t + db_t


_payload_tc = pl.pallas_call(
    _payload_tc_body,
    grid=(NL, NBLK),
    in_specs=[
        pl.BlockSpec((1, 1, ET), lambda l, eb: (eb, 0, 0)),
        pl.BlockSpec((NH, ET), lambda l, eb: (0, eb)),
        pl.BlockSpec((1, NH, BINS), lambda l, eb: (l, 0, 0)),
        pl.BlockSpec((1, H, BINS), lambda l, eb: (l, 0, 0)),
        pl.BlockSpec((1, 1, H), lambda l, eb: (l, 0, 0)),
        pl.BlockSpec((1, NH, H), lambda l, eb: (l, 0, 0)),
        pl.BlockSpec((1, 1, NH), lambda l, eb: (l, 0, 0)),
        pl.BlockSpec((1, NH, NH), lambda l, eb: (l, 0, 0)),
    ],
    out_specs=pl.BlockSpec((1, 2 * NH, ET), lambda l, eb: (l, 0, eb)),
    out_shape=jax.ShapeDtypeStruct((NL, 2 * NH, E_PAD), jnp.float32),
)


def _gat_pre_body(h_ref, w_ref, asrc_ref, adst_ref, sel_ref, tsrc_ref, tdst_ref):
    xp = h_ref[...] @ w_ref[...]
    a_s = (xp * asrc_ref[...]) @ sel_ref[...]
    a_d = (xp * adst_ref[...]) @ sel_ref[...]
    tsrc_ref[:, 0:H] = xp
    tsrc_ref[:, H:WSC] = a_s[:, 0:WSC - H]
    tdst_ref[...] = a_d[:, 0:16]


_gat_pre = pl.pallas_call(
    _gat_pre_body,
    grid=(NNB,),
    in_specs=[
        pl.BlockSpec((NT, H), lambda i: (i, 0)),
        pl.BlockSpec((H, H), lambda i: (0, 0)),
        pl.BlockSpec((1, H), lambda i: (0, 0)),
        pl.BlockSpec((1, H), lambda i: (0, 0)),
        pl.BlockSpec((H, H), lambda i: (0, 0)),
    ],
    out_specs=[
        pl.BlockSpec((NT, WSC), lambda i: (i, 0)),
        pl.BlockSpec((NT, 16), lambda i: (i, 0)),
    ],
    out_shape=[
        jax.ShapeDtypeStruct((N_PAD, WSC), jnp.float32),
        jax.ShapeDtypeStruct((N_PAD, 16), jnp.float32),
    ],
)


def _norm_agg(acc0, acc1):
    acc = acc0 + acc1
    num = acc[:, 0:H]
    den = acc[:, H:H + NH]
    rec = 1.0 / (den + 1e-16)
    return num * jnp.repeat(rec, HD, axis=1)


def _layer_norm(h):
    mu = jnp.mean(h, axis=-1, keepdims=True)
    var = jnp.mean((h - mu) ** 2, axis=-1, keepdims=True)
    return (h - mu) / jnp.sqrt(var + 1e-5)


def _gat_post_body(h_ref, a0_ref, a1_ref, bias_ref, out_ref):
    aggv = _norm_agg(a0_ref[0], a1_ref[0])
    hn = h_ref[...] + aggv + bias_ref[...]
    out_ref[...] = jax.nn.silu(_layer_norm(hn))


_gat_post = pl.pallas_call(
    _gat_post_body,
    grid=(NNB,),
    in_specs=[
        pl.BlockSpec((NT, H), lambda i: (i, 0)),
        pl.BlockSpec((1, NT, WSC), lambda i: (0, i, 0)),
        pl.BlockSpec((1, NT, WSC), lambda i: (1, i, 0)),
        pl.BlockSpec((1, H), lambda i: (0, 0)),
    ],
    out_specs=pl.BlockSpec((NT, H), lambda i: (i, 0)),
    out_shape=jax.ShapeDtypeStruct((N_PAD, H), jnp.float32),
)


def _attn_pre_body(h_ref, qw_ref, qb_ref, kw_ref, kb_ref, vw_ref, vb_ref,
                   tsrc_ref, tdst_ref):
    h_ = h_ref[...]
    tsrc_ref[:, 0:H] = h_ @ qw_ref[0] + qb_ref[...]
    tsrc_ref[:, H:2 * H] = h_ @ vw_ref[0] + vb_ref[...]
    tdst_ref[...] = h_ @ kw_ref[0] + kb_ref[...]


_attn_pre = pl.pallas_call(
    _attn_pre_body,
    grid=(NNB,),
    in_specs=[
        pl.BlockSpec((NT, H), lambda i: (i, 0)),
        pl.BlockSpec((1, H, H), lambda i: (0, 0, 0)),
        pl.BlockSpec((1, H), lambda i: (0, 0)),
        pl.BlockSpec((1, H, H), lambda i: (0, 0, 0)),
        pl.BlockSpec((1, H), lambda i: (0, 0)),
        pl.BlockSpec((1, H, H), lambda i: (0, 0, 0)),
        pl.BlockSpec((1, H), lambda i: (0, 0)),
    ],
    out_specs=[
        pl.BlockSpec((NT, 2 * H), lambda i: (i, 0)),
        pl.BlockSpec((NT, H), lambda i: (i, 0)),
    ],
    out_shape=[
        jax.ShapeDtypeStruct((N_PAD, 2 * H), jnp.float32),
        jax.ShapeDtypeStruct((N_PAD, H), jnp.float32),
    ],
)


def _attn_post_body(h_ref, a0_ref, a1_ref, ow_ref, ob_ref, out_ref):
    aggv = _norm_agg(a0_ref[0], a1_ref[0])
    out = aggv @ ow_ref[0] + ob_ref[...]
    out_ref[...] = _layer_norm(h_ref[...] + out)


_attn_post = pl.pallas_call(
    _attn_post_body,
    grid=(NNB,),
    in_specs=[
        pl.BlockSpec((NT, H), lambda i: (i, 0)),
        pl.BlockSpec((1, NT, WSC), lambda i: (0, i, 0)),
        pl.BlockSpec((1, NT, WSC), lambda i: (1, i, 0)),
        pl.BlockSpec((1, H, H), lambda i: (0, 0, 0)),
        pl.BlockSpec((1, H), lambda i: (0, 0)),
    ],
    out_specs=pl.BlockSpec((NT, H), lambda i: (i, 0)),
    out_shape=jax.ShapeDtypeStruct((N_PAD, H), jnp.float32),
)


def _pool_head_body(oh_ref, h_ref, w1_ref, b1_ref, w2_ref, b2_ref, o_ref,
                    gsum_ref, cnt_ref):
    i = pl.program_id(0)

    @pl.when(i == 0)
    def _():
        gsum_ref[...] = jnp.zeros_like(gsum_ref)
        cnt_ref[...] = jnp.zeros_like(cnt_ref)

    oh = oh_ref[...]
    dn = (((0,), (0,)), ((), ()))
    gsum_ref[...] += lax.dot_general(oh, h_ref[...], dn)
    cnt_ref[...] += lax.dot_general(oh, jnp.ones((NT, H), jnp.float32), dn)

    @pl.when(i == NNB - 1)
    def _():
        g = gsum_ref[...] / jnp.maximum(cnt_ref[...], 1.0)
        z = jax.nn.silu(g @ w1_ref[...] + b1_ref[...][None, :])
        o_ref[...] = z @ w2_ref[...] + b2_ref[...][None, :]


_pool_head = pl.pallas_call(
    _pool_head_body,
    grid=(NNB,),
    in_specs=[
        pl.BlockSpec((NT, NG), lambda i: (i, 0)),
        pl.BlockSpec((NT, H), lambda i: (i, 0)),
        pl.BlockSpec((H, H), lambda i: (0, 0)),
        pl.BlockSpec((H,), lambda i: (0,)),
        pl.BlockSpec((H, 1), lambda i: (0, 0)),
        pl.BlockSpec((1,), lambda i: (0,)),
    ],
    out_specs=pl.BlockSpec((NG, 1), lambda i: (0, 0)),
    out_shape=jax.ShapeDtypeStruct((NG, 1), jnp.float32),
    scratch_shapes=[
        pltpu.VMEM((NG, H), jnp.float32),
        pltpu.VMEM((NG, H), jnp.float32),
    ],
)

_SEL = None


def _sel_matrix():
    global _SEL
    if _SEL is None:
        import numpy as _np
        s = _np.zeros((H, H), _np.float32)
        for k in range(H):
            if k // HD < NH:
                s[k, k // HD] = 1.0
        _SEL = jnp.asarray(s)
    return _SEL


def kernel(x, is_defect, edge_index, edge_attr, batch, atom_emb, defect_emb, gat_W, gat_att_src, gat_att_dst, gat_lin_edge, gat_att_edge, gat_bias, q_W, q_b, k_W, k_b, v_W, v_b, o_W, o_b, geo_W1, geo_b1, geo_W2, geo_b2, defect_bias, fc1_W, fc1_b, fc2_W, fc2_b):
    src = edge_index[0]
    dst = edge_index[1]
    srci = jnp.pad(src, (0, E_PAD - E)).astype(jnp.int32)
    dsti = jnp.pad(dst, (0, E_PAD - E)).astype(jnp.int32)
    zrows = jnp.zeros((STRIPE, WSC), jnp.float32)

    h = atom_emb[x] + defect_emb[is_defect]

    attr_r = jnp.pad(edge_attr[:, 0], (0, E_PAD - E)).reshape(NBLK, 1, ET)
    code = is_defect[src] * 2 + is_defect[dst]
    oh = (jnp.pad(code, (0, E_PAD - E))[None, :]
          == jnp.arange(NH, dtype=code.dtype)[:, None]).astype(jnp.float32)
    mt = jnp.einsum("lbnd,lnd->lnb", gat_lin_edge.reshape(NL, BINS, NH, HD),
                    gat_att_edge)
    w1t = jnp.transpose(geo_W1, (0, 2, 1))
    w2t = jnp.transpose(geo_W2, (0, 2, 1))
    pay = _payload_tc(attr_r, oh, mt, w1t, geo_b1.reshape(NL, 1, H), w2t,
                      geo_b2.reshape(NL, 1, NH), defect_bias)
    pay = jnp.where(
        (jnp.arange(E_PAD) < E)[None, None, :], pay, -1e30)

    sel = _sel_matrix()
    for l in range(NL):
        tsrc, tdst = _gat_pre(h, gat_W[l], gat_att_src[l].reshape(1, H),
                              gat_att_dst[l].reshape(1, H), sel)
        out = _gat_edge(tsrc, tdst, pay[l], srci, dsti, zrows)
        h = _gat_post(h, out, out, gat_bias[l].reshape(1, H))

    for l in range(NL):
        tsrc, tdst = _attn_pre(h, q_W[l:l + 1], q_b[l].reshape(1, H),
                               k_W[l:l + 1], k_b[l].reshape(1, H),
                               v_W[l:l + 1], v_b[l].reshape(1, H))
        out = _attn_edge(tsrc, tdst, pay[l], srci, dsti, zrows)
        h = _attn_post(h, out, out, o_W[l:l + 1], o_b[l].reshape(1, H))

    ohb = (batch[:, None] == jnp.arange(NG, dtype=batch.dtype)[None, :]
           ).astype(jnp.float32)
    return _pool_head(ohb, h, fc1_W, fc1_b, fc2_W, fc2_b)


# speedup vs baseline: 11.7467x; 1.0419x over previous
import functools
import math

import jax
import jax.numpy as jnp
from jax import lax
from jax.experimental import pallas as pl
from jax.experimental.pallas import tpu as pltpu
from jax.experimental.pallas import tpu_sc as plsc

N = 10000
E = 320000
H = 128
NH = 4
HD = 32
NL = 3
NG = 32
BINS = 40

NCORES = 2
NSUB = 16
NW = NCORES * NSUB
N_PAD = 10000
STRIPE = N_PAD // NSUB
EPW = 10240
E_PAD = EPW * NW
WSC = 144

_MESH = plsc.VectorSubcoreMesh(core_axis_name="c", subcore_axis_name="s")


def _iota16():
    return lax.iota(jnp.int32, 16)


def _splat(v):
    return jnp.full((16,), v, jnp.int32)


def _edge_kernel_body(mode, c_sz, poff, tsrc, tdst, pay, srci, dsti, zrows, out,
                      agg, rows, kd, pay_v, e_buf, src_v, dst_v, sc_v,
                      sem, sem2, sem3, sem4, sem5):
    cid = lax.axis_index("c")
    sid = lax.axis_index("s")
    wid = cid * NSUB + sid
    nchunk = EPW // c_sz
    ngroup = c_sz // 16

    pltpu.sync_copy(zrows, agg.at[pl.ds(sid * STRIPE, STRIPE)])
    if mode == "attn":
        pltpu.sync_copy(zrows.at[pl.ds(0, c_sz)], sc_v)
    plsc.subcore_barrier()

    sc = rows if mode == "gat" else sc_v
    voff = 0 if mode == "gat" else H

    def chunk(ci, _):
        base = wid * EPW + ci * c_sz
        ic1 = pltpu.async_copy(srci.at[pl.ds(base, c_sz)], src_v, sem3)
        ic2 = pltpu.async_copy(dsti.at[pl.ds(base, c_sz)], dst_v, sem4)
        ic3 = pltpu.async_copy(pay.at[:, pl.ds(base, c_sz)], pay_v, sem5)
        ic1.wait()
        ic2.wait()
        cp1 = pltpu.async_copy(tsrc.at[src_v], rows, sem)
        cp2 = pltpu.async_copy(tdst.at[dst_v], kd, sem2)
        ic3.wait()
        cp1.wait()
        cp2.wait()

        for g in range(ngroup):
            g16 = g * 16 + _iota16()
            for h in range(NH):
                pay_h = pay_v[poff + h, pl.ds(g * 16, 16)]
                if mode == "gat":
                    asrc = plsc.load_gather(rows, [g16, _splat(H + h)])
                    adst = plsc.load_gather(kd, [g16, _splat(h)])
                    logit = asrc + adst + pay_h
                    logit = jnp.maximum(logit, 0.2 * logit)
                    e_h = jnp.exp(logit)
                else:
                    acc = jnp.zeros((16,), jnp.float32)
                    for j in range(HD):
                        col = _splat(h * HD + j)
                        qc = plsc.load_gather(rows, [g16, col])
                        kc = plsc.load_gather(kd, [g16, col])
                        acc = acc + qc * kc
                    e_h = jnp.exp(acc * (1.0 / math.sqrt(HD)) + pay_h)
                plsc.store_scatter(sc, [g16, _splat(H + h)], e_h)
                e_buf[h, pl.ds(g * 16, 16)] = e_h

        def edge(c2, _):
            for u in range(2):
                cc = c2 * 2 + u
                for hp in range(NH):
                    es = plsc.load_gather(e_buf, [_splat(hp), _splat(cc)])
                    for k in (2 * hp, 2 * hp + 1):
                        v = rows[cc, pl.ds(voff + k * 16, 16)]
                        sc[cc, pl.ds(k * 16, 16)] = v * es
            return 0

        lax.fori_loop(0, c_sz // 2, edge, 0)

        pltpu.sync_copy(sc, agg.at[dst_v], add=True)
        return 0

    lax.fori_loop(0, nchunk, chunk, 0)
    plsc.subcore_barrier()
    pltpu.sync_copy(agg.at[pl.ds(sid * STRIPE, STRIPE)],
                    out.at[cid, pl.ds(sid * STRIPE, STRIPE)])


def _make_edge_kernel(mode, c_sz, poff, wsrc, wdst):
    body = functools.partial(_edge_kernel_body, mode, c_sz, poff)
    return pl.kernel(
        body,
        mesh=_MESH,
        out_type=jax.ShapeDtypeStruct((NCORES, N_PAD, WSC), jnp.float32),
        scratch_types=[
            pltpu.VMEM_SHARED((N_PAD, WSC), jnp.float32),
            pltpu.VMEM((c_sz, wsrc), jnp.float32),
            pltpu.VMEM((c_sz, wdst), jnp.float32),
            pltpu.VMEM((8, c_sz), jnp.float32),
            pltpu.VMEM((NH, c_sz), jnp.float32),
            pltpu.VMEM((c_sz,), jnp.int32),
            pltpu.VMEM((c_sz,), jnp.int32),
            pltpu.VMEM((c_sz if mode == "attn" else 16, WSC), jnp.float32),
            pltpu.SemaphoreType.DMA,
            pltpu.SemaphoreType.DMA,
            pltpu.SemaphoreType.DMA,
            pltpu.SemaphoreType.DMA,
            pltpu.SemaphoreType.DMA,
        ],
        compiler_params=pltpu.CompilerParams(use_tc_tiling_on_sc=False,
                                             needs_layout_passes=False),
    )


_gat_edge = _make_edge_kernel("gat", 128, 0, WSC, 16)
_attn_edge = _make_edge_kernel("attn", 64, 4, 2 * H, H)


ET = 4096
NBLK = E_PAD // ET
NT = 2000
NNB = N_PAD // NT


def _payload_tc_body(attr_ref, oh_ref, mt_ref, w1t_ref, b1_ref, w2t_ref,
                     b2_ref, db_ref, out_ref):
    attr = attr_ref[0, 0, :]
    cent = lax.broadcasted_iota(jnp.int32, (BINS, 1), 0).astype(jnp.float32) * (
        8.0 / (BINS - 1))
    ef_t = jnp.exp(-10.0 * (attr[None, :] - cent) ** 2)
    a_et = mt_ref[0] @ ef_t
    z = w1t_ref[0] @ ef_t + b1_ref[0, 0][:, None]
    geo_t = w2t_ref[0] @ jax.nn.silu(z) + b2_ref[0, 0][:, None]
    db_t = db_ref[0] @ oh_ref[...]
    out_ref[0, 0:NH, :] = a_et
    out_ref[0, NH:2 * NH, :] = geo_t + db_t


_payload_tc = pl.pallas_call(
    _payload_tc_body,
    grid=(NL, NBLK),
    in_specs=[
        pl.BlockSpec((1, 1, ET), lambda l, eb: (eb, 0, 0)),
        pl.BlockSpec((NH, ET), lambda l, eb: (0, eb)),
        pl.BlockSpec((1, NH, BINS), lambda l, eb: (l, 0, 0)),
        pl.BlockSpec((1, H, BINS), lambda l, eb: (l, 0, 0)),
        pl.BlockSpec((1, 1, H), lambda l, eb: (l, 0, 0)),
        pl.BlockSpec((1, NH, H), lambda l, eb: (l, 0, 0)),
        pl.BlockSpec((1, 1, NH), lambda l, eb: (l, 0, 0)),
        pl.BlockSpec((1, NH, NH), lambda l, eb: (l, 0, 0)),
    ],
    out_specs=pl.BlockSpec((1, 2 * NH, ET), lambda l, eb: (l, 0, eb)),
    out_shape=jax.ShapeDtypeStruct((NL, 2 * NH, E_PAD), jnp.float32),
)


def _gat_pre_body(h_ref, w_ref, asrc_ref, adst_ref, sel_ref, tsrc_ref, tdst_ref):
    xp = h_ref[...] @ w_ref[...]
    a_s = (xp * asrc_ref[...]) @ sel_ref[...]
    a_d = (xp * adst_ref[...]) @ sel_ref[...]
    tsrc_ref[:, 0:H] = xp
    tsrc_ref[:, H:WSC] = a_s[:, 0:WSC - H]
    tdst_ref[...] = a_d[:, 0:16]


_gat_pre = pl.pallas_call(
    _gat_pre_body,
    grid=(NNB,),
    in_specs=[
        pl.BlockSpec((NT, H), lambda i: (i, 0)),
        pl.BlockSpec((H, H), lambda i: (0, 0)),
        pl.BlockSpec((1, H), lambda i: (0, 0)),
        pl.BlockSpec((1, H), lambda i: (0, 0)),
        pl.BlockSpec((H, H), lambda i: (0, 0)),
    ],
    out_specs=[
        pl.BlockSpec((NT, WSC), lambda i: (i, 0)),
        pl.BlockSpec((NT, 16), lambda i: (i, 0)),
    ],
    out_shape=[
        jax.ShapeDtypeStruct((N_PAD, WSC), jnp.float32),
        jax.ShapeDtypeStruct((N_PAD, 16), jnp.float32),
    ],
)


def _norm_agg(acc0, acc1):
    acc = acc0 + acc1
    num = acc[:, 0:H]
    den = acc[:, H:H + NH]
    rec = 1.0 / (den + 1e-16)
    return num * jnp.repeat(rec, HD, axis=1)


def _layer_norm(h):
    mu = jnp.mean(h, axis=-1, keepdims=True)
    var = jnp.mean((h - mu) ** 2, axis=-1, keepdims=True)
    return (h - mu) / jnp.sqrt(var + 1e-5)


def _gat_post_body(h_ref, a0_ref, a1_ref, bias_ref, out_ref):
    aggv = _norm_agg(a0_ref[0], a1_ref[0])
    hn = h_ref[...] + aggv + bias_ref[...]
    out_ref[...] = jax.nn.silu(_layer_norm(hn))


_gat_post = pl.pallas_call(
    _gat_post_body,
    grid=(NNB,),
    in_specs=[
        pl.BlockSpec((NT, H), lambda i: (i, 0)),
        pl.BlockSpec((1, NT, WSC), lambda i: (0, i, 0)),
        pl.BlockSpec((1, NT, WSC), lambda i: (1, i, 0)),
        pl.BlockSpec((1, H), lambda i: (0, 0)),
    ],
    out_specs=pl.BlockSpec((NT, H), lambda i: (i, 0)),
    out_shape=jax.ShapeDtypeStruct((N_PAD, H), jnp.float32),
)


def _attn_pre_body(h_ref, qw_ref, qb_ref, kw_ref, kb_ref, vw_ref, vb_ref,
                   tsrc_ref, tdst_ref):
    h_ = h_ref[...]
    tsrc_ref[:, 0:H] = h_ @ qw_ref[0] + qb_ref[...]
    tsrc_ref[:, H:2 * H] = h_ @ vw_ref[0] + vb_ref[...]
    tdst_ref[...] = h_ @ kw_ref[0] + kb_ref[...]


_attn_pre = pl.pallas_call(
    _attn_pre_body,
    grid=(NNB,),
    in_specs=[
        pl.BlockSpec((NT, H), lambda i: (i, 0)),
        pl.BlockSpec((1, H, H), lambda i: (0, 0, 0)),
        pl.BlockSpec((1, H), lambda i: (0, 0)),
        pl.BlockSpec((1, H, H), lambda i: (0, 0, 0)),
        pl.BlockSpec((1, H), lambda i: (0, 0)),
        pl.BlockSpec((1, H, H), lambda i: (0, 0, 0)),
        pl.BlockSpec((1, H), lambda i: (0, 0)),
    ],
    out_specs=[
        pl.BlockSpec((NT, 2 * H), lambda i: (i, 0)),
        pl.BlockSpec((NT, H), lambda i: (i, 0)),
    ],
    out_shape=[
        jax.ShapeDtypeStruct((N_PAD, 2 * H), jnp.float32),
        jax.ShapeDtypeStruct((N_PAD, H), jnp.float32),
    ],
)


def _attn_post_body(h_ref, a0_ref, a1_ref, ow_ref, ob_ref, out_ref):
    aggv = _norm_agg(a0_ref[0], a1_ref[0])
    out = aggv @ ow_ref[0] + ob_ref[...]
    out_ref[...] = _layer_norm(h_ref[...] + out)


_attn_post = pl.pallas_call(
    _attn_post_body,
    grid=(NNB,),
    in_specs=[
        pl.BlockSpec((NT, H), lambda i: (i, 0)),
        pl.BlockSpec((1, NT, WSC), lambda i: (0, i, 0)),
        pl.BlockSpec((1, NT, WSC), lambda i: (1, i, 0)),
        pl.BlockSpec((1, H, H), lambda i: (0, 0, 0)),
        pl.BlockSpec((1, H), lambda i: (0, 0)),
    ],
    out_specs=pl.BlockSpec((NT, H), lambda i: (i, 0)),
    out_shape=jax.ShapeDtypeStruct((N_PAD, H), jnp.float32),
)


def _pool_head_body(oh_ref, h_ref, w1_ref, b1_ref, w2_ref, b2_ref, o_ref,
                    gsum_ref, cnt_ref):
    i = pl.program_id(0)

    @pl.when(i == 0)
    def _():
        gsum_ref[...] = jnp.zeros_like(gsum_ref)
        cnt_ref[...] = jnp.zeros_like(cnt_ref)

    oh = oh_ref[...]
    dn = (((0,), (0,)), ((), ()))
    gsum_ref[...] += lax.dot_general(oh, h_ref[...], dn)
    cnt_ref[...] += lax.dot_general(oh, jnp.ones((NT, H), jnp.float32), dn)

    @pl.when(i == NNB - 1)
    def _():
        g = gsum_ref[...] / jnp.maximum(cnt_ref[...], 1.0)
        z = jax.nn.silu(g @ w1_ref[...] + b1_ref[...][None, :])
        o_ref[...] = z @ w2_ref[...] + b2_ref[...][None, :]


_pool_head = pl.pallas_call(
    _pool_head_body,
    grid=(NNB,),
    in_specs=[
        pl.BlockSpec((NT, NG), lambda i: (i, 0)),
        pl.BlockSpec((NT, H), lambda i: (i, 0)),
        pl.BlockSpec((H, H), lambda i: (0, 0)),
        pl.BlockSpec((H,), lambda i: (0,)),
        pl.BlockSpec((H, 1), lambda i: (0, 0)),
        pl.BlockSpec((1,), lambda i: (0,)),
    ],
    out_specs=pl.BlockSpec((NG, 1), lambda i: (0, 0)),
    out_shape=jax.ShapeDtypeStruct((NG, 1), jnp.float32),
    scratch_shapes=[
        pltpu.VMEM((NG, H), jnp.float32),
        pltpu.VMEM((NG, H), jnp.float32),
    ],
)

_SEL = None


def _sel_matrix():
    global _SEL
    if _SEL is None:
        import numpy as _np
        s = _np.zeros((H, H), _np.float32)
        for k in range(H):
            if k // HD < NH:
                s[k, k // HD] = 1.0
        _SEL = jnp.asarray(s)
    return _SEL


def kernel(x, is_defect, edge_index, edge_attr, batch, atom_emb, defect_emb, gat_W, gat_att_src, gat_att_dst, gat_lin_edge, gat_att_edge, gat_bias, q_W, q_b, k_W, k_b, v_W, v_b, o_W, o_b, geo_W1, geo_b1, geo_W2, geo_b2, defect_bias, fc1_W, fc1_b, fc2_W, fc2_b):
    src = edge_index[0]
    dst = edge_index[1]
    srci = jnp.pad(src, (0, E_PAD - E)).astype(jnp.int32)
    dsti = jnp.pad(dst, (0, E_PAD - E)).astype(jnp.int32)
    zrows = jnp.zeros((STRIPE, WSC), jnp.float32)

    h = atom_emb[x] + defect_emb[is_defect]

    attr_r = jnp.pad(edge_attr[:, 0], (0, E_PAD - E)).reshape(NBLK, 1, ET)
    code = is_defect[src] * 2 + is_defect[dst]
    oh = (jnp.pad(code, (0, E_PAD - E))[None, :]
          == jnp.arange(NH, dtype=code.dtype)[:, None]).astype(jnp.float32)
    mt = jnp.einsum("lbnd,lnd->lnb", gat_lin_edge.reshape(NL, BINS, NH, HD),
                    gat_att_edge)
    w1t = jnp.transpose(geo_W1, (0, 2, 1))
    w2t = jnp.transpose(geo_W2, (0, 2, 1))
    pay = _payload_tc(attr_r, oh, mt, w1t, geo_b1.reshape(NL, 1, H), w2t,
                      geo_b2.reshape(NL, 1, NH), defect_bias)
    pay = jnp.where(
        (jnp.arange(E_PAD) < E)[None, None, :], pay, -1e30)

    sel = _sel_matrix()
    for l in range(NL):
        tsrc, tdst = _gat_pre(h, gat_W[l], gat_att_src[l].reshape(1, H),
                              gat_att_dst[l].reshape(1, H), sel)
        out = _gat_edge(tsrc, tdst, pay[l], srci, dsti, zrows)
        h = _gat_post(h, out, out, gat_bias[l].reshape(1, H))

    for l in range(NL):
        tsrc, tdst = _attn_pre(h, q_W[l:l + 1], q_b[l].reshape(1, H),
                               k_W[l:l + 1], k_b[l].reshape(1, H),
                               v_W[l:l + 1], v_b[l].reshape(1, H))
        out = _attn_edge(tsrc, tdst, pay[l], srci, dsti, zrows)
        h = _attn_post(h, out, out, o_W[l:l + 1], o_b[l].reshape(1, H))

    ohb = (batch[:, None] == jnp.arange(NG, dtype=batch.dtype)[None, :]
           ).astype(jnp.float32)
    return _pool_head(ohb, h, fc1_W, fc1_b, fc2_W, fc2_b)
